# Initial kernel scaffold; baseline (speedup 1.0000x reference)
import jax
import jax.numpy as jnp
from jax.experimental import pallas as pl


def kernel(x, pos_table):
    raise NotImplementedError("write your pallas kernel here")


# speedup vs baseline: 1.6838x; 1.6838x over previous
import jax
import jax.numpy as jnp
from jax.experimental import pallas as pl

SEQ_TILE = 1024


def _add_kernel(x_ref, pos_ref, o_ref):
    o_ref[...] = x_ref[...] + pos_ref[...]


def kernel(x, pos_table):
    batch, seq, dim = x.shape
    n_seq = seq // SEQ_TILE
    return pl.pallas_call(
        _add_kernel,
        grid=(n_seq, batch),
        in_specs=[
            pl.BlockSpec((1, SEQ_TILE, dim), lambda s, b: (b, s, 0)),
            pl.BlockSpec((SEQ_TILE, dim), lambda s, b: (s, 0)),
        ],
        out_specs=pl.BlockSpec((1, SEQ_TILE, dim), lambda s, b: (b, s, 0)),
        out_shape=jax.ShapeDtypeStruct((batch, seq, dim), x.dtype),
    )(x, pos_table)


# speedup vs baseline: 1.7927x; 1.0647x over previous
import jax
import jax.numpy as jnp
from jax.experimental import pallas as pl

SEQ_TILE = 2048


def _add_kernel(x_ref, pos_ref, o_ref):
    o_ref[...] = x_ref[...] + pos_ref[...]


def kernel(x, pos_table):
    batch, seq, dim = x.shape
    n_seq = seq // SEQ_TILE
    return pl.pallas_call(
        _add_kernel,
        grid=(n_seq, batch),
        in_specs=[
            pl.BlockSpec((1, SEQ_TILE, dim), lambda s, b: (b, s, 0)),
            pl.BlockSpec((SEQ_TILE, dim), lambda s, b: (s, 0)),
        ],
        out_specs=pl.BlockSpec((1, SEQ_TILE, dim), lambda s, b: (b, s, 0)),
        out_shape=jax.ShapeDtypeStruct((batch, seq, dim), x.dtype),
    )(x, pos_table)


# speedup vs baseline: 1.8111x; 1.0103x over previous
import jax
import jax.numpy as jnp
from jax.experimental import pallas as pl

SEQ_TILE = 1024


def _add_kernel(x_ref, pos_ref, o_ref):
    o_ref[...] = x_ref[...] + pos_ref[...]


def kernel(x, pos_table):
    batch, seq, dim = x.shape
    n_seq = seq // SEQ_TILE
    return pl.pallas_call(
        _add_kernel,
        grid=(n_seq,),
        in_specs=[
            pl.BlockSpec((batch, SEQ_TILE, dim), lambda s: (0, s, 0)),
            pl.BlockSpec((SEQ_TILE, dim), lambda s: (s, 0)),
        ],
        out_specs=pl.BlockSpec((batch, SEQ_TILE, dim), lambda s: (0, s, 0)),
        out_shape=jax.ShapeDtypeStruct((batch, seq, dim), x.dtype),
    )(x, pos_table)
